# Initial kernel scaffold; baseline (speedup 1.0000x reference)
import jax
import jax.numpy as jnp
from jax.experimental import pallas as pl


def kernel(lang, table):
    raise NotImplementedError("write your pallas kernel here")


# speedup vs baseline: 2.5782x; 2.5782x over previous
import functools

import jax
import jax.numpy as jnp
from jax import lax
from jax.experimental import pallas as pl
from jax.experimental.pallas import tpu as pltpu
from jax.experimental.pallas import tpu_sc as plsc

BATCH = 16384
BAG = 50
DIM = 64
NC = 2
NS = 16
NW = NC * NS
BPW = BATCH // NW
CB = 2
RPG = CB * BAG
NCHUNK = BPW // CB
LANES = 16
KD = DIM // LANES


def _bag_sum(rows_ref, row_base):
    def body(r, acc):
        row = row_base + r
        return tuple(
            acc[k] + rows_ref[row, pl.ds(k * LANES, LANES)] for k in range(KD)
        )
    init = tuple(jnp.zeros((LANES,), jnp.float32) for _ in range(KD))
    return lax.fori_loop(0, BAG, body, init, unroll=2)


def _embedding_bag_mean(lang3, table):
    mesh = plsc.VectorSubcoreMesh(core_axis_name="c", subcore_axis_name="s")

    @functools.partial(
        pl.kernel,
        out_type=jax.ShapeDtypeStruct((BATCH, DIM), jnp.float32),
        mesh=mesh,
        compiler_params=pltpu.CompilerParams(use_tc_tiling_on_sc=False),
        scratch_types=[
            pltpu.VMEM((NCHUNK, RPG), jnp.int32),
            pltpu.VMEM((RPG, DIM), jnp.float32),
            pltpu.VMEM((RPG, DIM), jnp.float32),
            pltpu.VMEM((BPW, DIM), jnp.float32),
            pltpu.SemaphoreType.DMA,
            pltpu.SemaphoreType.DMA,
        ],
    )
    def kern(lang_hbm, table_hbm, out_hbm, idx_all, rows0, rows1, out_v,
             sem0, sem1):
        wid = lax.axis_index("s") * NC + lax.axis_index("c")
        rows = (rows0, rows1)
        sems = (sem0, sem1)
        scale = jnp.float32(1.0 / BAG)

        pltpu.sync_copy(lang_hbm.at[wid], idx_all)

        for b in range(2):
            pltpu.async_copy(table_hbm.at[idx_all.at[b]], rows[b], sems[b])

        def chunk_body(i, _):
            for b in range(2):
                g = 2 * i + b
                pltpu.make_async_copy(
                    table_hbm.at[idx_all.at[0]], rows[b], sems[b]
                ).wait()
                for c in range(CB):
                    acc = _bag_sum(rows[b], c * BAG)
                    bag = g * CB + c
                    for k in range(KD):
                        out_v[bag, pl.ds(k * LANES, LANES)] = acc[k] * scale

                @pl.when(g + 2 < NCHUNK)
                def _():
                    pltpu.async_copy(
                        table_hbm.at[idx_all.at[g + 2]], rows[b], sems[b]
                    )
            return 0

        lax.fori_loop(0, NCHUNK // 2, chunk_body, 0)

        pltpu.sync_copy(out_v, out_hbm.at[pl.ds(wid * BPW, BPW)])

    return kern(lang3, table)


def kernel(lang, table):
    idx = lang.astype(jnp.int32).reshape(NW, NCHUNK, RPG)
    return _embedding_bag_mean(idx, table)


# speedup vs baseline: 2.7970x; 1.0848x over previous
import functools

import jax
import jax.numpy as jnp
from jax import lax
from jax.experimental import pallas as pl
from jax.experimental.pallas import tpu as pltpu
from jax.experimental.pallas import tpu_sc as plsc

BATCH = 16384
BAG = 50
DIM = 64
NC = 2
NS = 16
NW = NC * NS
BPW = BATCH // NW
CB = 8
RPG = CB * BAG
NCHUNK = BPW // CB
LANES = 16
KD = DIM // LANES


def _bag_sum(rows_ref, row_base):
    def body(r, acc):
        row = row_base + r
        return tuple(
            acc[k] + rows_ref[row, pl.ds(k * LANES, LANES)] for k in range(KD)
        )
    init = tuple(jnp.zeros((LANES,), jnp.float32) for _ in range(KD))
    return lax.fori_loop(0, BAG, body, init, unroll=2)


def _embedding_bag_mean(lang3, table):
    mesh = plsc.VectorSubcoreMesh(core_axis_name="c", subcore_axis_name="s")

    @functools.partial(
        pl.kernel,
        out_type=jax.ShapeDtypeStruct((BATCH, DIM), jnp.float32),
        mesh=mesh,
        compiler_params=pltpu.CompilerParams(use_tc_tiling_on_sc=False),
        scratch_types=[
            pltpu.VMEM((NCHUNK, RPG), jnp.int32),
            pltpu.VMEM((RPG, DIM), jnp.float32),
            pltpu.VMEM((RPG, DIM), jnp.float32),
            pltpu.VMEM((BPW, DIM), jnp.float32),
            pltpu.SemaphoreType.DMA,
            pltpu.SemaphoreType.DMA,
        ],
    )
    def kern(lang_hbm, table_hbm, out_hbm, idx_all, rows0, rows1, out_v,
             sem0, sem1):
        wid = lax.axis_index("s") * NC + lax.axis_index("c")
        rows = (rows0, rows1)
        sems = (sem0, sem1)
        scale = jnp.float32(1.0 / BAG)

        pltpu.sync_copy(lang_hbm.at[wid], idx_all)

        for b in range(2):
            pltpu.async_copy(table_hbm.at[idx_all.at[b]], rows[b], sems[b])

        def chunk_body(i, _):
            for b in range(2):
                g = 2 * i + b
                pltpu.make_async_copy(
                    table_hbm.at[idx_all.at[0]], rows[b], sems[b]
                ).wait()
                for c in range(CB):
                    acc = _bag_sum(rows[b], c * BAG)
                    bag = g * CB + c
                    for k in range(KD):
                        out_v[bag, pl.ds(k * LANES, LANES)] = acc[k] * scale

                @pl.when(g + 2 < NCHUNK)
                def _():
                    pltpu.async_copy(
                        table_hbm.at[idx_all.at[g + 2]], rows[b], sems[b]
                    )
            return 0

        lax.fori_loop(0, NCHUNK // 2, chunk_body, 0)

        pltpu.sync_copy(out_v, out_hbm.at[pl.ds(wid * BPW, BPW)])

    return kern(lang3, table)


def kernel(lang, table):
    idx = lang.astype(jnp.int32).reshape(NW, NCHUNK, RPG)
    return _embedding_bag_mean(idx, table)
